# Initial kernel scaffold; baseline (speedup 1.0000x reference)
import jax
import jax.numpy as jnp
from jax.experimental import pallas as pl


def kernel(vol, trf):
    raise NotImplementedError("write your pallas kernel here")


# speedup vs baseline: 1.7554x; 1.7554x over previous
import functools

import jax
import jax.numpy as jnp
from jax import lax
from jax.experimental import pallas as pl
from jax.experimental.pallas import tpu as pltpu
from jax.experimental.pallas import tpu_sc as plsc

B, H, W, C = 4, 224, 224, 96
CP = 128
HW = H * W
N = B * HW
NW = 32
PPW = N // NW
K = 128
NCHUNK = PPW // K
G = K // 16
CG = C // 16


def _st_body(vol_hbm, ty_hbm, tx_hbm, out_hbm,
             ty_v, tx_v,
             i00, i01, i10, i11,
             w00, w01, w10, w11,
             r00, r01, r10, r11,
             gsem):
  cid = lax.axis_index("c")
  sid = lax.axis_index("s")
  wid = sid * 2 + cid
  pix0 = wid * PPW
  vol_base = (pix0 // HW) * HW

  def chunk_body(g, carry):
    start = pix0 + g * K
    pltpu.sync_copy(ty_hbm.at[pl.ds(start, K)], ty_v)
    pltpu.sync_copy(tx_hbm.at[pl.ds(start, K)], tx_v)

    for j in range(G):
      sl = pl.ds(j * 16, 16)
      ty = ty_v[sl]
      tx = tx_v[sl]
      y0t = ty.astype(jnp.int32)
      y0t = jnp.where(y0t.astype(jnp.float32) > ty, y0t - 1, y0t)
      x0t = tx.astype(jnp.int32)
      x0t = jnp.where(x0t.astype(jnp.float32) > tx, x0t - 1, x0t)
      y0 = jnp.clip(y0t, 0, H - 1)
      y1 = jnp.clip(y0t + 1, 0, H - 1)
      x0 = jnp.clip(x0t, 0, W - 1)
      x1 = jnp.clip(x0t + 1, 0, W - 1)
      tyc = jnp.clip(ty, 0.0, float(H - 1))
      txc = jnp.clip(tx, 0.0, float(W - 1))
      wy0 = y1.astype(jnp.float32) - tyc
      wy1 = 1.0 - wy0
      wx0 = x1.astype(jnp.float32) - txc
      wx1 = 1.0 - wx0
      yb0 = vol_base + y0 * W
      yb1 = vol_base + y1 * W
      i00[sl] = yb0 + x0
      i01[sl] = yb0 + x1
      i10[sl] = yb1 + x0
      i11[sl] = yb1 + x1
      w00[sl] = wy0 * wx0
      w01[sl] = wy0 * wx1
      w10[sl] = wy1 * wx0
      w11[sl] = wy1 * wx1

    c0 = pltpu.async_copy(vol_hbm.at[i00], r00, gsem)
    c1 = pltpu.async_copy(vol_hbm.at[i01], r01, gsem)
    c2 = pltpu.async_copy(vol_hbm.at[i10], r10, gsem)
    c3 = pltpu.async_copy(vol_hbm.at[i11], r11, gsem)
    c0.wait()
    c1.wait()
    c2.wait()
    c3.wait()

    def grp_body(jj, c):
      sl = pl.ds(jj * 16, 16)
      wv00 = w00[sl]
      wv01 = w01[sl]
      wv10 = w10[sl]
      wv11 = w11[sl]
      base = jj * 16
      for i in range(16):
        p = base + i
        a00 = wv00[i]
        a01 = wv01[i]
        a10 = wv10[i]
        a11 = wv11[i]
        for cg in range(CG):
          s2 = pl.ds(cg * 16, 16)
          acc = a00 * r00[p, s2]
          acc = acc + a01 * r01[p, s2]
          acc = acc + a10 * r10[p, s2]
          acc = acc + a11 * r11[p, s2]
          r00[p, s2] = acc
      return c

    lax.fori_loop(0, G, grp_body, 0, unroll=False)

    pltpu.sync_copy(r00, out_hbm.at[pl.ds(start, K)])
    return carry

  lax.fori_loop(0, NCHUNK, chunk_body, 0, unroll=False)


@functools.partial(
    pl.kernel,
    mesh=plsc.VectorSubcoreMesh(core_axis_name="c", subcore_axis_name="s"),
    out_type=jax.ShapeDtypeStruct((N, CP), jnp.float32),
    scratch_types=[
        pltpu.VMEM((K,), jnp.float32),
        pltpu.VMEM((K,), jnp.float32),
        pltpu.VMEM((K,), jnp.int32),
        pltpu.VMEM((K,), jnp.int32),
        pltpu.VMEM((K,), jnp.int32),
        pltpu.VMEM((K,), jnp.int32),
        pltpu.VMEM((K,), jnp.float32),
        pltpu.VMEM((K,), jnp.float32),
        pltpu.VMEM((K,), jnp.float32),
        pltpu.VMEM((K,), jnp.float32),
        pltpu.VMEM((K, CP), jnp.float32),
        pltpu.VMEM((K, CP), jnp.float32),
        pltpu.VMEM((K, CP), jnp.float32),
        pltpu.VMEM((K, CP), jnp.float32),
        pltpu.SemaphoreType.DMA,
    ],
)
def _st_kernel(vol_hbm, ty_hbm, tx_hbm, out_hbm, *rest):
  _st_body(vol_hbm, ty_hbm, tx_hbm, out_hbm, *rest)


def kernel(vol, trf):
  vol_p = jnp.pad(vol.reshape(N, C), ((0, 0), (0, CP - C)))
  ty = trf[..., 0].reshape(N)
  tx = trf[..., 1].reshape(N)
  out = _st_kernel(vol_p, ty, tx)
  return out[:, :C].reshape(B, H, W, C)
